# baseline (speedup 1.0000x reference)
import functools

import jax
import jax.numpy as jnp
from jax import lax
from jax.experimental import pallas as pl
from jax.experimental.pallas import tpu as pltpu
from jax.experimental.pallas import tpu_sc as plsc

B = 16384
D = 64
NC = 2
NS = 16
NW = NC * NS
BPW = B // NW
CHUNK = 128
NCHUNK = BPW // CHUNK
L = 16
HALF = BPW // 2


def _sc_gather_dot(u_idx3, g_idx3, cat_emb, user_bias1, game_bias1):
    mesh = plsc.VectorSubcoreMesh(core_axis_name="c", subcore_axis_name="s")

    @functools.partial(
        pl.kernel,
        mesh=mesh,
        compiler_params=pltpu.CompilerParams(use_tc_tiling_on_sc=True),
        out_type=[
            jax.ShapeDtypeStruct((NW, L), jnp.float32),
            jax.ShapeDtypeStruct((B,), jnp.float32),
            jax.ShapeDtypeStruct((B,), jnp.float32),
        ],
        scratch_types=[
            pltpu.VMEM((NCHUNK, CHUNK), jnp.int32),
            pltpu.VMEM((NCHUNK, CHUNK), jnp.int32),
            pltpu.VMEM((HALF, 2 * D), jnp.float32),
            pltpu.VMEM((HALF, 2 * D), jnp.float32),
            pltpu.VMEM((BPW,), jnp.float32),
            pltpu.VMEM((BPW,), jnp.float32),
            pltpu.VMEM((L,), jnp.float32),
            pltpu.SemaphoreType.DMA,
            pltpu.SemaphoreType.DMA,
            pltpu.SemaphoreType.DMA,
            pltpu.SemaphoreType.DMA,
        ],
    )
    def k(uidx_hbm, gidx_hbm, cat_hbm, ubias_hbm, gbias_hbm,
          part_out, ub_out, gb_out,
          uidx_v, gidx_v, urows, grows, ub_v, gb_v, acc_v,
          sem_u, sem_g, sem_ub, sem_gb):
        wid = lax.axis_index("s") * NC + lax.axis_index("c")
        base = wid * BPW

        pltpu.sync_copy(uidx_hbm.at[wid], uidx_v)
        pltpu.sync_copy(gidx_hbm.at[wid], gidx_v)

        bias_copies = []
        for j in range(NCHUNK):
            rows = pl.ds(j * CHUNK, CHUNK)
            bias_copies.append(pltpu.async_copy(
                ubias_hbm.at[uidx_v.at[j]], ub_v.at[rows], sem_ub))
            bias_copies.append(pltpu.async_copy(
                gbias_hbm.at[gidx_v.at[j]], gb_v.at[rows], sem_gb))

        z = jnp.zeros((L,), jnp.float32)
        accs = (z, z, z, z)
        for h in range(2):
            copies = []
            for jj in range(NCHUNK // 2):
                j = h * (NCHUNK // 2) + jj
                rows = pl.ds(jj * CHUNK, CHUNK)
                copies.append(pltpu.async_copy(
                    cat_hbm.at[uidx_v.at[j]], urows.at[rows], sem_u))
                copies.append(pltpu.async_copy(
                    cat_hbm.at[gidx_v.at[j]], grows.at[rows], sem_g))
            for c in copies:
                c.wait()

            def body(i, accs):
                a0, a1, a2, a3 = accs
                a0 = a0 + urows[i, pl.ds(0, 16)] * grows[i, pl.ds(64, 16)]
                a1 = a1 + urows[i, pl.ds(16, 16)] * grows[i, pl.ds(80, 16)]
                a2 = a2 + urows[i, pl.ds(32, 16)] * grows[i, pl.ds(96, 16)]
                a3 = a3 + urows[i, pl.ds(48, 16)] * grows[i, pl.ds(112, 16)]
                return (a0, a1, a2, a3)

            accs = lax.fori_loop(0, HALF, body, accs)

        a0, a1, a2, a3 = accs
        acc_v[...] = (a0 + a1) + (a2 + a3)
        pltpu.sync_copy(acc_v, part_out.at[wid])

        for c in bias_copies:
            c.wait()
        pltpu.sync_copy(ub_v, ub_out.at[pl.ds(base, BPW)])
        pltpu.sync_copy(gb_v, gb_out.at[pl.ds(base, BPW)])

    return k(u_idx3, g_idx3, cat_emb, user_bias1, game_bias1)


def _tc_finalize(partials, ub2, gb2):
    def body(p_ref, u_ref, g_ref, o_ref):
        s = jnp.sum(p_ref[...])
        o_ref[...] = jax.nn.sigmoid(s + u_ref[...] + g_ref[...])

    return pl.pallas_call(
        body,
        out_shape=jax.ShapeDtypeStruct(ub2.shape, jnp.float32),
    )(partials, ub2, gb2)


def kernel(inputs, user_emb, user_bias_table, game_emb, game_bias_table):
    u_idx = inputs[:, 0].astype(jnp.int32)
    g_idx = inputs[:, 1].astype(jnp.int32)
    u3 = u_idx.reshape(NW, NCHUNK, CHUNK)
    g3 = g_idx.reshape(NW, NCHUNK, CHUNK)
    cat = jnp.zeros((user_emb.shape[0], 2 * D), jnp.float32)
    cat = lax.dynamic_update_slice(cat, user_emb, (0, 0))
    cat = lax.dynamic_update_slice(cat, game_emb, (0, D))
    partials, ub, gb = _sc_gather_dot(
        u3, g3, cat,
        user_bias_table.reshape(-1), game_bias_table.reshape(-1))
    out = _tc_finalize(partials, ub.reshape(128, 128), gb.reshape(128, 128))
    return out.reshape(B, 1)


# speedup vs baseline: 3.1407x; 3.1407x over previous
import functools

import jax
import jax.numpy as jnp
from jax import lax
from jax.experimental import pallas as pl
from jax.experimental.pallas import tpu as pltpu
from jax.experimental.pallas import tpu_sc as plsc

B = 16384
D = 64
NC = 2
NS = 16
NW = NC * NS
BPW = B // NW
CHUNK = 128
NCHUNK = BPW // CHUNK
L = 16
HALF = BPW // 2


def _sc_gather_dot(u_idx3, g_idx3, cat_emb, user_bias1, game_bias1):
    mesh = plsc.VectorSubcoreMesh(core_axis_name="c", subcore_axis_name="s")

    @functools.partial(
        pl.kernel,
        mesh=mesh,
        compiler_params=pltpu.CompilerParams(use_tc_tiling_on_sc=True),
        out_type=[
            jax.ShapeDtypeStruct((NW, L), jnp.float32),
            jax.ShapeDtypeStruct((B,), jnp.float32),
            jax.ShapeDtypeStruct((B,), jnp.float32),
        ],
        scratch_types=[
            pltpu.VMEM((NCHUNK, CHUNK), jnp.int32),
            pltpu.VMEM((NCHUNK, CHUNK), jnp.int32),
            pltpu.VMEM((HALF, 2 * D), jnp.float32),
            pltpu.VMEM((HALF, 2 * D), jnp.float32),
            pltpu.VMEM((BPW,), jnp.float32),
            pltpu.VMEM((BPW,), jnp.float32),
            pltpu.VMEM((L,), jnp.float32),
            pltpu.SemaphoreType.DMA,
            pltpu.SemaphoreType.DMA,
            pltpu.SemaphoreType.DMA,
            pltpu.SemaphoreType.DMA,
        ],
    )
    def k(uidx_hbm, gidx_hbm, cat_hbm, ubias_hbm, gbias_hbm,
          part_out, ub_out, gb_out,
          uidx_v, gidx_v, urows, grows, ub_v, gb_v, acc_v,
          sem_u, sem_g, sem_ub, sem_gb):
        wid = lax.axis_index("s") * NC + lax.axis_index("c")
        base = wid * BPW

        pltpu.sync_copy(uidx_hbm.at[wid], uidx_v)
        pltpu.sync_copy(gidx_hbm.at[wid], gidx_v)

        bias_copies = []
        for j in range(NCHUNK):
            rows = pl.ds(j * CHUNK, CHUNK)
            bias_copies.append(pltpu.async_copy(
                ubias_hbm.at[uidx_v.at[j]], ub_v.at[rows], sem_ub))
            bias_copies.append(pltpu.async_copy(
                gbias_hbm.at[gidx_v.at[j]], gb_v.at[rows], sem_gb))

        z = jnp.zeros((L,), jnp.float32)
        accs = (z, z, z, z)
        for h in range(2):
            copies = []
            for jj in range(NCHUNK // 2):
                j = h * (NCHUNK // 2) + jj
                rows = pl.ds(jj * CHUNK, CHUNK)
                copies.append(pltpu.async_copy(
                    cat_hbm.at[uidx_v.at[j]], urows.at[rows], sem_u))
                copies.append(pltpu.async_copy(
                    cat_hbm.at[gidx_v.at[j]], grows.at[rows], sem_g))
            for c in copies:
                c.wait()

            def body(i, accs):
                a0, a1, a2, a3 = accs
                a0 = a0 + urows[i, pl.ds(0, 16)] * grows[i, pl.ds(64, 16)]
                a1 = a1 + urows[i, pl.ds(16, 16)] * grows[i, pl.ds(80, 16)]
                a2 = a2 + urows[i, pl.ds(32, 16)] * grows[i, pl.ds(96, 16)]
                a3 = a3 + urows[i, pl.ds(48, 16)] * grows[i, pl.ds(112, 16)]
                return (a0, a1, a2, a3)

            accs = lax.fori_loop(0, HALF, body, accs)

        a0, a1, a2, a3 = accs
        acc_v[...] = (a0 + a1) + (a2 + a3)
        pltpu.sync_copy(acc_v, part_out.at[wid])

        for c in bias_copies:
            c.wait()
        pltpu.sync_copy(ub_v, ub_out.at[pl.ds(base, BPW)])
        pltpu.sync_copy(gb_v, gb_out.at[pl.ds(base, BPW)])

    return k(u_idx3, g_idx3, cat_emb, user_bias1, game_bias1)


def _tc_finalize(partials, ub2, gb2):
    def body(p_ref, u_ref, g_ref, o_ref):
        s = jnp.sum(p_ref[...])
        o_ref[...] = jax.nn.sigmoid(s + u_ref[...] + g_ref[...])

    return pl.pallas_call(
        body,
        out_shape=jax.ShapeDtypeStruct(ub2.shape, jnp.float32),
    )(partials, ub2, gb2)


def kernel(inputs, user_emb, user_bias_table, game_emb, game_bias_table):
    u_idx = inputs[:, 0].astype(jnp.int32)
    g_idx = inputs[:, 1].astype(jnp.int32)
    u3 = u_idx.reshape(NW, NCHUNK, CHUNK)
    g3 = g_idx.reshape(NW, NCHUNK, CHUNK)
    catT = lax.optimization_barrier(
        jnp.concatenate([user_emb.T, game_emb.T], axis=0))
    cat = catT.T
    partials, ub, gb = _sc_gather_dot(
        u3, g3, cat,
        user_bias_table.reshape(-1), game_bias_table.reshape(-1))
    out = _tc_finalize(partials, ub.reshape(128, 128), gb.reshape(128, 128))
    return out.reshape(B, 1)


# speedup vs baseline: 3.1551x; 1.0046x over previous
import functools

import jax
import jax.numpy as jnp
from jax import lax
from jax.experimental import pallas as pl
from jax.experimental.pallas import tpu as pltpu
from jax.experimental.pallas import tpu_sc as plsc

B = 16384
D = 64
NC = 2
NS = 16
NW = NC * NS
BPW = B // NW
CHUNK = 128
NCHUNK = BPW // CHUNK
L = 16
HALF = BPW // 2


def _sc_gather_dot(u_idx3, g_idx3, cat_emb, user_bias1, game_bias1):
    mesh = plsc.VectorSubcoreMesh(core_axis_name="c", subcore_axis_name="s")

    @functools.partial(
        pl.kernel,
        mesh=mesh,
        compiler_params=pltpu.CompilerParams(use_tc_tiling_on_sc=True),
        out_type=[
            jax.ShapeDtypeStruct((NW, L), jnp.float32),
            jax.ShapeDtypeStruct((B,), jnp.float32),
            jax.ShapeDtypeStruct((B,), jnp.float32),
        ],
        scratch_types=[
            pltpu.VMEM((NCHUNK, CHUNK), jnp.int32),
            pltpu.VMEM((NCHUNK, CHUNK), jnp.int32),
            pltpu.VMEM((HALF, 2 * D), jnp.float32),
            pltpu.VMEM((HALF, 2 * D), jnp.float32),
            pltpu.VMEM((BPW,), jnp.float32),
            pltpu.VMEM((BPW,), jnp.float32),
            pltpu.VMEM((L,), jnp.float32),
            pltpu.SemaphoreType.DMA,
            pltpu.SemaphoreType.DMA,
            pltpu.SemaphoreType.DMA,
            pltpu.SemaphoreType.DMA,
        ],
    )
    def k(uidx_hbm, gidx_hbm, cat_hbm, ubias_hbm, gbias_hbm,
          part_out, ub_out, gb_out,
          uidx_v, gidx_v, urows, grows, ub_v, gb_v, acc_v,
          sem_u, sem_g, sem_ub, sem_gb):
        wid = lax.axis_index("s") * NC + lax.axis_index("c")
        base = wid * BPW

        pltpu.sync_copy(uidx_hbm.at[wid], uidx_v)
        pltpu.sync_copy(gidx_hbm.at[wid], gidx_v)

        bias_copies = []
        for j in range(NCHUNK):
            rows = pl.ds(j * CHUNK, CHUNK)
            bias_copies.append(pltpu.async_copy(
                ubias_hbm.at[uidx_v.at[j]], ub_v.at[rows], sem_ub))
            bias_copies.append(pltpu.async_copy(
                gbias_hbm.at[gidx_v.at[j]], gb_v.at[rows], sem_gb))

        z = jnp.zeros((L,), jnp.float32)
        accs = (z, z, z, z)
        for h in range(2):
            copies = []
            for jj in range(NCHUNK // 2):
                j = h * (NCHUNK // 2) + jj
                rows = pl.ds(jj * CHUNK, CHUNK)
                copies.append(pltpu.async_copy(
                    cat_hbm.at[uidx_v.at[j]], urows.at[rows], sem_u))
                copies.append(pltpu.async_copy(
                    cat_hbm.at[gidx_v.at[j]], grows.at[rows], sem_g))
            for c in copies:
                c.wait()

            def body(i, accs):
                a0, a1, a2, a3 = accs
                a0 = a0 + urows[i, pl.ds(0, 16)] * grows[i, pl.ds(64, 16)]
                a1 = a1 + urows[i, pl.ds(16, 16)] * grows[i, pl.ds(80, 16)]
                a2 = a2 + urows[i, pl.ds(32, 16)] * grows[i, pl.ds(96, 16)]
                a3 = a3 + urows[i, pl.ds(48, 16)] * grows[i, pl.ds(112, 16)]
                return (a0, a1, a2, a3)

            accs = lax.fori_loop(0, HALF, body, accs)

        a0, a1, a2, a3 = accs
        acc_v[...] = (a0 + a1) + (a2 + a3)
        pltpu.sync_copy(acc_v, part_out.at[wid])

        for c in bias_copies:
            c.wait()
        pltpu.sync_copy(ub_v, ub_out.at[pl.ds(base, BPW)])
        pltpu.sync_copy(gb_v, gb_out.at[pl.ds(base, BPW)])

    return k(u_idx3, g_idx3, cat_emb, user_bias1, game_bias1)


def _tc_finalize(partials, ub2, gb2):
    def body(p_ref, u_ref, g_ref, o_ref):
        s = jnp.sum(p_ref[...])
        o_ref[...] = jax.nn.sigmoid(s + u_ref[...] + g_ref[...])

    return pl.pallas_call(
        body,
        out_shape=jax.ShapeDtypeStruct(ub2.shape, jnp.float32),
    )(partials, ub2, gb2)


def kernel(inputs, user_emb, user_bias_table, game_emb, game_bias_table):
    u_idx = inputs[:, 0].astype(jnp.int32)
    g_idx = inputs[:, 1].astype(jnp.int32)
    u3 = u_idx.reshape(NW, NCHUNK, CHUNK)
    g3 = g_idx.reshape(NW, NCHUNK, CHUNK)
    cat = jnp.concatenate([user_emb, game_emb], axis=1)
    partials, ub, gb = _sc_gather_dot(
        u3, g3, cat,
        user_bias_table.reshape(-1), game_bias_table.reshape(-1))
    out = _tc_finalize(partials, ub.reshape(128, 128), gb.reshape(128, 128))
    return out.reshape(B, 1)
